# baseline (speedup 1.0000x reference)
import jax
import jax.numpy as jnp
from jax.experimental import pallas as pl

_SIZE = 32000
_SMOOTHING = 0.1
_CONFIDENCE = 1.0 - _SMOOTHING
_EPS = _SMOOTHING / (_SIZE - 2)

_ROWS = 8192
_BLOCK_R = 128


def _smooth_kernel(tgt_ref, out_ref):
    i = pl.program_id(0)
    tgt = tgt_ref[i, :]
    cols = jax.lax.broadcasted_iota(jnp.int32, (_BLOCK_R, _SIZE), 1)
    out_ref[:, :] = jnp.where(
        cols == tgt[:, None],
        jnp.float32(_CONFIDENCE),
        jnp.float32(_EPS),
    )


def kernel(target):
    nb = _ROWS // _BLOCK_R
    tgt2 = target.astype(jnp.int32).reshape(nb, _BLOCK_R)
    out = pl.pallas_call(
        _smooth_kernel,
        grid=(nb,),
        in_specs=[pl.BlockSpec((nb, _BLOCK_R), lambda i: (0, 0))],
        out_specs=pl.BlockSpec((_BLOCK_R, _SIZE), lambda i: (i, 0)),
        out_shape=jax.ShapeDtypeStruct((_ROWS, _SIZE), jnp.float32),
    )(tgt2)
    return out


# speedup vs baseline: 1.0282x; 1.0282x over previous
import jax
import jax.numpy as jnp
from jax.experimental import pallas as pl

_SIZE = 32000
_SMOOTHING = 0.1
_CONFIDENCE = 1.0 - _SMOOTHING
_EPS = _SMOOTHING / (_SIZE - 2)

_ROWS = 8192
_BLOCK_R = 128


def _smooth_kernel(tgt_ref, out_ref):
    tgt = tgt_ref[0, 0, :]
    cols = jax.lax.broadcasted_iota(jnp.int32, (_BLOCK_R, _SIZE), 1)
    out_ref[:, :] = jnp.where(
        cols == tgt[:, None],
        jnp.float32(_CONFIDENCE),
        jnp.float32(_EPS),
    )


def kernel(target):
    nb = _ROWS // _BLOCK_R
    tgt3 = target.astype(jnp.int32).reshape(nb, 1, _BLOCK_R)
    out = pl.pallas_call(
        _smooth_kernel,
        grid=(nb,),
        in_specs=[pl.BlockSpec((1, 1, _BLOCK_R), lambda i: (i, 0, 0))],
        out_specs=pl.BlockSpec((_BLOCK_R, _SIZE), lambda i: (i, 0)),
        out_shape=jax.ShapeDtypeStruct((_ROWS, _SIZE), jnp.float32),
    )(tgt3)
    return out
